# Initial kernel scaffold; baseline (speedup 1.0000x reference)
import jax
import jax.numpy as jnp
from jax.experimental import pallas as pl


def kernel(geometry_data, gaussian_pcd, features, k):
    raise NotImplementedError("write your pallas kernel here")


# speedup vs baseline: 17.7015x; 17.7015x over previous
import functools
import jax
import jax.numpy as jnp
from jax.experimental import pallas as pl


def _body(q_ref, p_ref, f_ref, o_ref, *, K):
    TN = q_ref.shape[1]
    M = p_ref.shape[2]
    q = q_ref[0]
    p = p_ref[0]
    q2 = q[:, 0:1] ** 2 + q[:, 1:2] ** 2 + q[:, 2:3] ** 2
    p2 = p[0:1, :] ** 2 + p[1:2, :] ** 2 + p[2:3, :] ** 2
    qb = q.astype(jnp.bfloat16).astype(jnp.float32)
    pb = p.astype(jnp.bfloat16).astype(jnp.float32)
    qp = (qb[:, 0:1] * pb[0:1, :]
          + qb[:, 1:2] * pb[1:2, :]
          + qb[:, 2:3] * pb[2:3, :])
    d2 = (q2 + p2) - 2.0 * qp
    d = jnp.sqrt(jnp.maximum(d2, 1e-12))
    lane = jax.lax.broadcasted_iota(jnp.int32, (TN, M), 1)
    W = jnp.zeros((TN, M), jnp.float32)
    for _ in range(K):
        v = jnp.min(d, axis=1, keepdims=True)
        idx = jnp.argmin(d, axis=1)
        onehot = lane == idx[:, None]
        W = jnp.where(onehot, 1.0 / (v + 1e-8), W)
        d = jnp.where(onehot, jnp.float32(3e38), d)
    wsum = jnp.sum(W, axis=1, keepdims=True)
    o_ref[0] = jnp.dot(W, f_ref[0],
                       preferred_element_type=jnp.float32) / wsum


def kernel(geometry_data, gaussian_pcd, features, k):
    B, N, _ = geometry_data.shape
    M = gaussian_pcd.shape[1]
    C = features.shape[2]
    K = 10
    TN = 256
    pcd_t = jnp.swapaxes(gaussian_pcd, 1, 2)
    out = pl.pallas_call(
        functools.partial(_body, K=K),
        grid=(B, N // TN),
        in_specs=[
            pl.BlockSpec((1, TN, 3), lambda b, n: (b, n, 0)),
            pl.BlockSpec((1, 3, M), lambda b, n: (b, 0, 0)),
            pl.BlockSpec((1, M, C), lambda b, n: (b, 0, 0)),
        ],
        out_shape=jax.ShapeDtypeStruct((B, N, C), jnp.float32),
        out_specs=pl.BlockSpec((1, TN, C), lambda b, n: (b, n, 0)),
    )(geometry_data, pcd_t, features)
    return out
